# Initial kernel scaffold; baseline (speedup 1.0000x reference)
import jax
import jax.numpy as jnp
from jax.experimental import pallas as pl


def kernel(logits, dst):
    raise NotImplementedError("write your pallas kernel here")


# speedup vs baseline: 2.6415x; 2.6415x over previous
import jax
import jax.numpy as jnp
from jax import lax
from jax.experimental import pallas as pl
from jax.experimental.pallas import tpu as pltpu
from jax.experimental.pallas import tpu_sc as plsc

N_NODES = 100000
N_EDGES = 3200000
N_HEADS = 4

NC = 2
NS = 16
NW = NC * NS
L = 16

RN = 3128
NNP = NW * RN
RN4 = RN * N_HEADS

B = 512
NGRP = B // L
NCHUNK = B * N_HEADS // 128

EPAD = N_EDGES + 2 * B
DLEN = EPAD + 16

NEG = -3.0e38
DUMMY_ELT = (N_EDGES + B) * N_HEADS


def _lanes():
  return lax.broadcasted_iota(jnp.int32, (L,), 0)


def _worker_bounds(ebv, wid):
  aw = ebv[pl.ds(pl.multiple_of(wid * L, L), L)][0]
  ew = ebv[pl.ds(pl.multiple_of((NW + wid) * L, L), L)][0]
  return aw, ew


def _group_masks(dstv, e0, base_node):
  d0 = dstv[pl.ds(8 + e0, L)]
  dnx = dstv[pl.ds(8 + e0 + 1, L)]
  lanes = _lanes()
  bmask = (d0 != dnx) | (lanes == L - 1)
  nd = d0 - base_node
  valid = (nd >= 0) & (nd < RN)
  ndc = jnp.clip(nd, 0, RN - 1)
  return d0, bmask, valid, ndc


def _max_kernel(lgf_hbm, dsth_hbm, eb_hbm, maxh, lgb, dstv, msc, lm, ebv):
  wid = lax.axis_index("c") * NS + lax.axis_index("s")
  pltpu.sync_copy(eb_hbm, ebv)
  aw, ew = _worker_bounds(ebv, wid)
  nb = (ew - aw + B - 1) // B
  base_node = wid * RN
  lanes = _lanes()
  lanes4 = lanes * N_HEADS

  def init_body(i, _):
    lm[pl.ds(pl.multiple_of(i * L, L), L)] = jnp.full((L,), NEG, jnp.float32)
    return 0

  lax.fori_loop(0, (RN4 + L) // L, init_body, 0)
  for a in range(2):
    for h in range(N_HEADS):
      msc[a, h, pl.ds(0, L)] = jnp.full((L,), NEG, jnp.float32)

  def block_body(j, _):
    s = pl.multiple_of(aw + j * B, 8)
    pltpu.sync_copy(lgf_hbm.at[pl.ds(pl.multiple_of(s * N_HEADS, 8), B * N_HEADS)], lgb)
    pltpu.sync_copy(dsth_hbm.at[pl.ds(s, B + L)], dstv)

    for g in range(NGRP):
      e0 = g * L
      for h in range(N_HEADS):
        x = plsc.load_gather(lgb, [lanes4 + (e0 * N_HEADS + h)])
        msc[0, h, pl.ds(8 + e0, L)] = x
    src = 0
    for k in (1, 2, 4, 8):
      dst_b = 1 - src
      for g in range(NGRP):
        e0 = g * L
        d0 = dstv[pl.ds(8 + e0, L)]
        gk = (d0 == dstv[pl.ds(8 + e0 - k, L)]) & (lanes >= k)
        for h in range(N_HEADS):
          m = msc[src, h, pl.ds(8 + e0, L)]
          msh = msc[src, h, pl.ds(8 + e0 - k, L)]
          msc[dst_b, h, pl.ds(8 + e0, L)] = jnp.where(
              gk, jnp.maximum(m, msh), m
          )
      src = dst_b
    for g in range(NGRP):
      e0 = g * L
      _, bmask, valid, ndc = _group_masks(dstv, e0, base_node)
      wmask = bmask & valid
      for h in range(N_HEADS):
        m = msc[src, h, pl.ds(8 + e0, L)]
        midx = ndc * N_HEADS + h
        old = plsc.load_gather(lm, [midx])
        plsc.store_scatter(lm, [midx], jnp.maximum(old, m), mask=wmask)
    return 0

  lax.fori_loop(0, nb, block_body, 0)

  def fix_body(i, _):
    o = pl.multiple_of(i * L, L)
    v = lm[pl.ds(o, L)]
    lm[pl.ds(o, L)] = jnp.where(v < -1.0e37, 0.0, v)
    return 0

  lax.fori_loop(0, RN4 // L, fix_body, 0)
  woff = pl.multiple_of(wid * RN4, 8)
  pltpu.sync_copy(lm.at[pl.ds(0, RN4)], maxh.at[pl.ds(woff, RN4)])


def _score_kernel(lgf_hbm, dsth_hbm, eb_hbm, maxh_hbm, scoresh, normh,
                  lgb, dstv, msc, lmv, ls, sb, idxsc, ebv, flag, sem):
  wid = lax.axis_index("c") * NS + lax.axis_index("s")
  pltpu.sync_copy(eb_hbm, ebv)
  aw, ew = _worker_bounds(ebv, wid)
  nb = (ew - aw + B - 1) // B
  base_node = wid * RN
  lanes = _lanes()
  lanes4 = lanes * N_HEADS
  econst = lanes // N_HEADS
  hconst = lanes % N_HEADS

  def init_body(i, _):
    ls[pl.ds(pl.multiple_of(i * L, L), L)] = jnp.zeros((L,), jnp.float32)
    return 0

  lax.fori_loop(0, (RN4 + L) // L, init_body, 0)
  woff = pl.multiple_of(wid * RN4, 8)
  pltpu.sync_copy(maxh_hbm.at[pl.ds(woff, RN4)], lmv.at[pl.ds(0, RN4)])

  def block_body(j, _):
    s = pl.multiple_of(aw + j * B, 8)
    pltpu.sync_copy(lgf_hbm.at[pl.ds(pl.multiple_of(s * N_HEADS, 8), B * N_HEADS)], lgb)
    pltpu.sync_copy(dsth_hbm.at[pl.ds(s, B + L)], dstv)

    zl = jnp.zeros((L,), jnp.int32)
    flag[pl.ds(0, L)] = zl
    for g in range(NGRP):
      e0 = g * L
      _, _, valid, ndc = _group_masks(dstv, e0, base_node)
      plsc.store_scatter(
          flag, [zl], jnp.ones((L,), jnp.int32), mask=jnp.logical_not(valid)
      )
      for h in range(N_HEADS):
        eltidx = lanes4 + (e0 * N_HEADS + h)
        x = plsc.load_gather(lgb, [eltidx])
        midx = ndc * N_HEADS + h
        mv = plsc.load_gather(lmv, [midx])
        sc = jnp.exp(x - mv)
        plsc.store_scatter(sb, [eltidx], sc)
        msc[0, h, pl.ds(8 + e0, L)] = sc
    src = 0
    for k in (1, 2, 4, 8):
      dst_b = 1 - src
      for g in range(NGRP):
        e0 = g * L
        d0 = dstv[pl.ds(8 + e0, L)]
        gk = (d0 == dstv[pl.ds(8 + e0 - k, L)]) & (lanes >= k)
        for h in range(N_HEADS):
          m = msc[src, h, pl.ds(8 + e0, L)]
          msh = msc[src, h, pl.ds(8 + e0 - k, L)]
          msc[dst_b, h, pl.ds(8 + e0, L)] = jnp.where(gk, m + msh, m)
      src = dst_b
    for g in range(NGRP):
      e0 = g * L
      _, bmask, valid, ndc = _group_masks(dstv, e0, base_node)
      wmask = bmask & valid
      for h in range(N_HEADS):
        m = msc[src, h, pl.ds(8 + e0, L)]
        plsc.addupdate_scatter(ls, [ndc * N_HEADS + h], m, mask=wmask)

    clean = flag[pl.ds(0, L)][0] == 0

    @pl.when(clean)
    def _clean():
      pltpu.sync_copy(sb, scoresh.at[pl.ds(pl.multiple_of(s * N_HEADS, 8), B * N_HEADS)])

    @pl.when(jnp.logical_not(clean))
    def _boundary():
      for c in range(NCHUNK):
        for q in range(128 // L):
          evec = econst + (c * (128 // N_HEADS) + q * (L // N_HEADS))
          d4 = plsc.load_gather(dstv, [evec + 8])
          nd4 = d4 - base_node
          v4 = (nd4 >= 0) & (nd4 < RN)
          gidx = (s + evec) * N_HEADS + hconst
          rowv = jnp.where(v4, gidx, DUMMY_ELT + wid * L + lanes)
          idxsc[c, pl.ds(q * L, L)] = rowv
      for c in range(NCHUNK):
        pltpu.async_copy(
            sb.at[pl.ds(c * 128, 128)], scoresh.at[idxsc.at[c]], sem
        ).wait()

    return 0

  lax.fori_loop(0, nb, block_body, 0)
  woff2 = pl.multiple_of(wid * RN4, 8)
  pltpu.sync_copy(ls.at[pl.ds(0, RN4)], normh.at[pl.ds(woff2, RN4)])


def _mesh():
  return plsc.VectorSubcoreMesh(
      core_axis_name="c", subcore_axis_name="s", num_cores=NC, num_subcores=NS
  )


_params = pltpu.CompilerParams(needs_layout_passes=False)

_max_call = pl.kernel(
    _max_kernel,
    out_type=jax.ShapeDtypeStruct((NNP * N_HEADS,), jnp.float32),
    mesh=_mesh(),
    compiler_params=_params,
    scratch_types=[
        pltpu.VMEM((B * N_HEADS,), jnp.float32),
        pltpu.VMEM((B + L,), jnp.int32),
        pltpu.VMEM((2, N_HEADS, B + L), jnp.float32),
        pltpu.VMEM((RN4 + L,), jnp.float32),
        pltpu.VMEM((NW * 2 * L + L,), jnp.int32),
    ],
)

_score_call = pl.kernel(
    _score_kernel,
    out_type=(
        jax.ShapeDtypeStruct((EPAD * N_HEADS,), jnp.float32),
        jax.ShapeDtypeStruct((NNP * N_HEADS,), jnp.float32),
    ),
    mesh=_mesh(),
    compiler_params=_params,
    scratch_types=[
        pltpu.VMEM((B * N_HEADS,), jnp.float32),
        pltpu.VMEM((B + L,), jnp.int32),
        pltpu.VMEM((2, N_HEADS, B + L), jnp.float32),
        pltpu.VMEM((RN4 + L,), jnp.float32),
        pltpu.VMEM((RN4 + L,), jnp.float32),
        pltpu.VMEM((B * N_HEADS,), jnp.float32),
        pltpu.VMEM((NCHUNK, 128), jnp.int32),
        pltpu.VMEM((NW * 2 * L + L,), jnp.int32),
        pltpu.VMEM((L,), jnp.int32),
        pltpu.SemaphoreType.DMA,
    ],
)


@jax.jit
def kernel(logits, dst):
  lgf = jnp.concatenate([
      logits.reshape(N_EDGES * N_HEADS),
      jnp.zeros(((EPAD - N_EDGES) * N_HEADS,), jnp.float32),
  ])
  dsth = jnp.concatenate([
      jnp.full((8,), -1, jnp.int32),
      dst,
      jnp.full((DLEN - 8 - N_EDGES,), N_NODES, jnp.int32),
  ])
  bounds = jnp.arange(1, NW + 1, dtype=jnp.int32) * RN
  e_hi = jnp.searchsorted(dst, bounds, side="left").astype(jnp.int32)
  e_lo = jnp.concatenate([jnp.zeros((1,), jnp.int32), e_hi[:-1]])
  aw = (e_lo // 8) * 8
  vals = jnp.concatenate([aw, e_hi])
  eb = jnp.concatenate(
      [
          jnp.concatenate(
              [vals[:, None], jnp.zeros((2 * NW, L - 1), jnp.int32)], axis=1
          ).reshape(2 * NW * L),
          jnp.zeros((L,), jnp.int32),
      ]
  )

  maxh = _max_call(lgf, dsth, eb)
  scoresh, normh = _score_call(lgf, dsth, eb, maxh)
  scores = scoresh[: N_EDGES * N_HEADS].reshape(N_EDGES, N_HEADS, 1)
  normalizer = normh[: N_NODES * N_HEADS].reshape(N_NODES, N_HEADS, 1)
  return scores, normalizer


# speedup vs baseline: 2.7011x; 1.0226x over previous
import jax
import jax.numpy as jnp
from jax import lax
from jax.experimental import pallas as pl
from jax.experimental.pallas import tpu as pltpu
from jax.experimental.pallas import tpu_sc as plsc

N_NODES = 100000
N_EDGES = 3200000
N_HEADS = 4

NC = 2
NS = 16
NW = NC * NS
L = 16

RN = 3128
NNP = NW * RN
RN4 = RN * N_HEADS

B = 512
NGRP = B // L
NCHUNK = B * N_HEADS // 128

EPAD = N_EDGES + 2 * B
DLEN = EPAD + 16

NEG = -3.0e38
DUMMY_ELT = (N_EDGES + B) * N_HEADS


def _lanes():
  return lax.broadcasted_iota(jnp.int32, (L,), 0)


def _worker_bounds(ebv, wid):
  aw = ebv[pl.ds(pl.multiple_of(wid * L, L), L)][0]
  ew = ebv[pl.ds(pl.multiple_of((NW + wid) * L, L), L)][0]
  return aw, ew


def _group_masks(dstv, e0, base_node):
  d0 = dstv[pl.ds(8 + e0, L)]
  dnx = dstv[pl.ds(8 + e0 + 1, L)]
  lanes = _lanes()
  bmask = (d0 != dnx) | (lanes == L - 1)
  nd = d0 - base_node
  valid = (nd >= 0) & (nd < RN)
  ndc = jnp.clip(nd, 0, RN - 1)
  return d0, bmask, valid, ndc


def _max_kernel(lgf_hbm, dsth_hbm, eb_hbm, maxh, lgb, dstv, msc, lm, ebv, sem):
  wid = lax.axis_index("c") * NS + lax.axis_index("s")
  pltpu.sync_copy(eb_hbm, ebv)
  aw, ew = _worker_bounds(ebv, wid)
  nb = (ew - aw + B - 1) // B
  base_node = wid * RN
  lanes = _lanes()
  lanes4 = lanes * N_HEADS

  def init_body(i, _):
    lm[pl.ds(pl.multiple_of(i * L, L), L)] = jnp.full((L,), NEG, jnp.float32)
    return 0

  lax.fori_loop(0, (RN4 + L) // L, init_body, 0)
  for a in range(2):
    for h in range(N_HEADS):
      msc[a, h, pl.ds(0, L)] = jnp.full((L,), NEG, jnp.float32)

  def block_body(j, _):
    s = pl.multiple_of(aw + j * B, 8)
    cp1 = pltpu.async_copy(
        lgf_hbm.at[pl.ds(pl.multiple_of(s * N_HEADS, 8), B * N_HEADS)], lgb, sem
    )
    cp2 = pltpu.async_copy(dsth_hbm.at[pl.ds(s, B + L)], dstv, sem)
    cp1.wait()
    cp2.wait()

    for g in range(NGRP):
      e0 = g * L
      for h in range(N_HEADS):
        x = plsc.load_gather(lgb, [lanes4 + (e0 * N_HEADS + h)])
        msc[0, h, pl.ds(8 + e0, L)] = x
    src = 0
    for k in (1, 2, 4, 8):
      dst_b = 1 - src
      for g in range(NGRP):
        e0 = g * L
        d0 = dstv[pl.ds(8 + e0, L)]
        gk = (d0 == dstv[pl.ds(8 + e0 - k, L)]) & (lanes >= k)
        for h in range(N_HEADS):
          m = msc[src, h, pl.ds(8 + e0, L)]
          msh = msc[src, h, pl.ds(8 + e0 - k, L)]
          msc[dst_b, h, pl.ds(8 + e0, L)] = jnp.where(
              gk, jnp.maximum(m, msh), m
          )
      src = dst_b
    for g in range(NGRP):
      e0 = g * L
      _, bmask, valid, ndc = _group_masks(dstv, e0, base_node)
      wmask = bmask & valid
      for h in range(N_HEADS):
        m = msc[src, h, pl.ds(8 + e0, L)]
        midx = ndc * N_HEADS + h
        old = plsc.load_gather(lm, [midx])
        plsc.store_scatter(lm, [midx], jnp.maximum(old, m), mask=wmask)
    return 0

  lax.fori_loop(0, nb, block_body, 0)

  def fix_body(i, _):
    o = pl.multiple_of(i * L, L)
    v = lm[pl.ds(o, L)]
    lm[pl.ds(o, L)] = jnp.where(v < -1.0e37, 0.0, v)
    return 0

  lax.fori_loop(0, RN4 // L, fix_body, 0)
  woff = pl.multiple_of(wid * RN4, 8)
  pltpu.sync_copy(lm.at[pl.ds(0, RN4)], maxh.at[pl.ds(woff, RN4)])


def _score_kernel(lgf_hbm, dsth_hbm, eb_hbm, maxh_hbm, scoresh, normh,
                  lgb, dstv, msc, lmv, ls, sb, idxsc, ebv, flag, sem):
  wid = lax.axis_index("c") * NS + lax.axis_index("s")
  pltpu.sync_copy(eb_hbm, ebv)
  aw, ew = _worker_bounds(ebv, wid)
  nb = (ew - aw + B - 1) // B
  base_node = wid * RN
  lanes = _lanes()
  lanes4 = lanes * N_HEADS
  econst = lanes // N_HEADS
  hconst = lanes % N_HEADS

  def init_body(i, _):
    ls[pl.ds(pl.multiple_of(i * L, L), L)] = jnp.zeros((L,), jnp.float32)
    return 0

  lax.fori_loop(0, (RN4 + L) // L, init_body, 0)
  woff = pl.multiple_of(wid * RN4, 8)
  pltpu.sync_copy(maxh_hbm.at[pl.ds(woff, RN4)], lmv.at[pl.ds(0, RN4)])

  def block_body(j, _):
    s = pl.multiple_of(aw + j * B, 8)
    cp1 = pltpu.async_copy(
        lgf_hbm.at[pl.ds(pl.multiple_of(s * N_HEADS, 8), B * N_HEADS)], lgb, sem
    )
    cp2 = pltpu.async_copy(dsth_hbm.at[pl.ds(s, B + L)], dstv, sem)
    cp1.wait()
    cp2.wait()

    zl = jnp.zeros((L,), jnp.int32)
    flag[pl.ds(0, L)] = zl
    for g in range(NGRP):
      e0 = g * L
      _, _, valid, ndc = _group_masks(dstv, e0, base_node)
      plsc.store_scatter(
          flag, [zl], jnp.ones((L,), jnp.int32), mask=jnp.logical_not(valid)
      )
      for h in range(N_HEADS):
        eltidx = lanes4 + (e0 * N_HEADS + h)
        x = plsc.load_gather(lgb, [eltidx])
        midx = ndc * N_HEADS + h
        mv = plsc.load_gather(lmv, [midx])
        sc = jnp.exp(x - mv)
        plsc.store_scatter(sb, [eltidx], sc)
        msc[0, h, pl.ds(8 + e0, L)] = sc
    src = 0
    for k in (1, 2, 4, 8):
      dst_b = 1 - src
      for g in range(NGRP):
        e0 = g * L
        d0 = dstv[pl.ds(8 + e0, L)]
        gk = (d0 == dstv[pl.ds(8 + e0 - k, L)]) & (lanes >= k)
        for h in range(N_HEADS):
          m = msc[src, h, pl.ds(8 + e0, L)]
          msh = msc[src, h, pl.ds(8 + e0 - k, L)]
          msc[dst_b, h, pl.ds(8 + e0, L)] = jnp.where(gk, m + msh, m)
      src = dst_b
    for g in range(NGRP):
      e0 = g * L
      _, bmask, valid, ndc = _group_masks(dstv, e0, base_node)
      wmask = bmask & valid
      for h in range(N_HEADS):
        m = msc[src, h, pl.ds(8 + e0, L)]
        plsc.addupdate_scatter(ls, [ndc * N_HEADS + h], m, mask=wmask)

    clean = flag[pl.ds(0, L)][0] == 0

    @pl.when(clean)
    def _clean():
      pltpu.sync_copy(sb, scoresh.at[pl.ds(pl.multiple_of(s * N_HEADS, 8), B * N_HEADS)])

    @pl.when(jnp.logical_not(clean))
    def _boundary():
      for c in range(NCHUNK):
        for q in range(128 // L):
          evec = econst + (c * (128 // N_HEADS) + q * (L // N_HEADS))
          d4 = plsc.load_gather(dstv, [evec + 8])
          nd4 = d4 - base_node
          v4 = (nd4 >= 0) & (nd4 < RN)
          gidx = (s + evec) * N_HEADS + hconst
          rowv = jnp.where(v4, gidx, DUMMY_ELT + wid * L + lanes)
          idxsc[c, pl.ds(q * L, L)] = rowv
      for c in range(NCHUNK):
        pltpu.async_copy(
            sb.at[pl.ds(c * 128, 128)], scoresh.at[idxsc.at[c]], sem
        ).wait()

    return 0

  lax.fori_loop(0, nb, block_body, 0)
  woff2 = pl.multiple_of(wid * RN4, 8)
  pltpu.sync_copy(ls.at[pl.ds(0, RN4)], normh.at[pl.ds(woff2, RN4)])


def _mesh():
  return plsc.VectorSubcoreMesh(
      core_axis_name="c", subcore_axis_name="s", num_cores=NC, num_subcores=NS
  )


_params = pltpu.CompilerParams(needs_layout_passes=False)

_max_call = pl.kernel(
    _max_kernel,
    out_type=jax.ShapeDtypeStruct((NNP * N_HEADS,), jnp.float32),
    mesh=_mesh(),
    compiler_params=_params,
    scratch_types=[
        pltpu.VMEM((B * N_HEADS,), jnp.float32),
        pltpu.VMEM((B + L,), jnp.int32),
        pltpu.VMEM((2, N_HEADS, B + L), jnp.float32),
        pltpu.VMEM((RN4 + L,), jnp.float32),
        pltpu.VMEM((NW * 2 * L + L,), jnp.int32),
        pltpu.SemaphoreType.DMA,
    ],
)

_score_call = pl.kernel(
    _score_kernel,
    out_type=(
        jax.ShapeDtypeStruct((EPAD * N_HEADS,), jnp.float32),
        jax.ShapeDtypeStruct((NNP * N_HEADS,), jnp.float32),
    ),
    mesh=_mesh(),
    compiler_params=_params,
    scratch_types=[
        pltpu.VMEM((B * N_HEADS,), jnp.float32),
        pltpu.VMEM((B + L,), jnp.int32),
        pltpu.VMEM((2, N_HEADS, B + L), jnp.float32),
        pltpu.VMEM((RN4 + L,), jnp.float32),
        pltpu.VMEM((RN4 + L,), jnp.float32),
        pltpu.VMEM((B * N_HEADS,), jnp.float32),
        pltpu.VMEM((NCHUNK, 128), jnp.int32),
        pltpu.VMEM((NW * 2 * L + L,), jnp.int32),
        pltpu.VMEM((L,), jnp.int32),
        pltpu.SemaphoreType.DMA,
    ],
)


@jax.jit
def kernel(logits, dst):
  lgf = jnp.concatenate([
      logits.reshape(N_EDGES * N_HEADS),
      jnp.zeros(((EPAD - N_EDGES) * N_HEADS,), jnp.float32),
  ])
  dsth = jnp.concatenate([
      jnp.full((8,), -1, jnp.int32),
      dst,
      jnp.full((DLEN - 8 - N_EDGES,), N_NODES, jnp.int32),
  ])
  bounds = jnp.arange(1, NW + 1, dtype=jnp.int32) * RN
  e_hi = jnp.searchsorted(dst, bounds, side="left").astype(jnp.int32)
  e_lo = jnp.concatenate([jnp.zeros((1,), jnp.int32), e_hi[:-1]])
  aw = (e_lo // 8) * 8
  vals = jnp.concatenate([aw, e_hi])
  eb = jnp.concatenate(
      [
          jnp.concatenate(
              [vals[:, None], jnp.zeros((2 * NW, L - 1), jnp.int32)], axis=1
          ).reshape(2 * NW * L),
          jnp.zeros((L,), jnp.int32),
      ]
  )

  maxh = _max_call(lgf, dsth, eb)
  scoresh, normh = _score_call(lgf, dsth, eb, maxh)
  scores = scoresh[: N_EDGES * N_HEADS].reshape(N_EDGES, N_HEADS, 1)
  normalizer = normh[: N_NODES * N_HEADS].reshape(N_NODES, N_HEADS, 1)
  return scores, normalizer


# speedup vs baseline: 2.8142x; 1.0419x over previous
import jax
import jax.numpy as jnp
from jax import lax
from jax.experimental import pallas as pl
from jax.experimental.pallas import tpu as pltpu
from jax.experimental.pallas import tpu_sc as plsc

N_NODES = 100000
N_EDGES = 3200000
N_HEADS = 4

NC = 2
NS = 16
NW = NC * NS
L = 16

RN = 3128
NNP = NW * RN
RN4 = RN * N_HEADS

B = 512
NGRP = B // L
NCHUNK = B * N_HEADS // 128

EPAD = N_EDGES + 2 * B
DLEN = EPAD + 16

NEG = -3.0e38
DUMMY_ELT = (N_EDGES + B) * N_HEADS


def _lanes():
  return lax.broadcasted_iota(jnp.int32, (L,), 0)


def _worker_bounds(ebv, wid):
  aw = ebv[pl.ds(pl.multiple_of(wid * L, L), L)][0]
  ew = ebv[pl.ds(pl.multiple_of((NW + wid) * L, L), L)][0]
  return aw, ew


def _group_masks(dstv, e0, base_node):
  d0 = dstv[pl.ds(8 + e0, L)]
  dnx = dstv[pl.ds(8 + e0 + 1, L)]
  lanes = _lanes()
  bmask = (d0 != dnx) | (lanes == L - 1)
  nd = d0 - base_node
  valid = (nd >= 0) & (nd < RN)
  ndc = jnp.clip(nd, 0, RN - 1)
  return d0, bmask, valid, ndc


def _max_kernel(lgf_hbm, dsth_hbm, eb_hbm, maxh, lgb, dstv, msc, lm, ebv, sem):
  wid = lax.axis_index("c") * NS + lax.axis_index("s")
  pltpu.sync_copy(eb_hbm, ebv)
  aw, ew = _worker_bounds(ebv, wid)
  nb = (ew - aw + B - 1) // B
  base_node = wid * RN
  lanes = _lanes()
  lanes4 = lanes * N_HEADS

  def init_body(i, _):
    lm[pl.ds(pl.multiple_of(i * L, L), L)] = jnp.full((L,), NEG, jnp.float32)
    return 0

  lax.fori_loop(0, (RN4 + L) // L, init_body, 0)
  for a in range(2):
    for h in range(N_HEADS):
      msc[a, h, pl.ds(0, L)] = jnp.full((L,), NEG, jnp.float32)

  def block_body(j, _):
    s = pl.multiple_of(aw + j * B, 8)
    cp1 = pltpu.async_copy(
        lgf_hbm.at[pl.ds(pl.multiple_of(s * N_HEADS, 8), B * N_HEADS)], lgb, sem
    )
    cp2 = pltpu.async_copy(dsth_hbm.at[pl.ds(s, B + L)], dstv, sem)
    cp1.wait()
    cp2.wait()

    for g in range(NGRP):
      e0 = g * L
      for h in range(N_HEADS):
        x = plsc.load_gather(lgb, [lanes4 + (e0 * N_HEADS + h)])
        msc[0, h, pl.ds(8 + e0, L)] = x
    src = 0
    for k in (1, 2, 4, 8):
      dst_b = 1 - src
      for g in range(NGRP):
        e0 = g * L
        d0 = dstv[pl.ds(8 + e0, L)]
        gk = (d0 == dstv[pl.ds(8 + e0 - k, L)]) & (lanes >= k)
        for h in range(N_HEADS):
          m = msc[src, h, pl.ds(8 + e0, L)]
          msh = msc[src, h, pl.ds(8 + e0 - k, L)]
          msc[dst_b, h, pl.ds(8 + e0, L)] = jnp.where(
              gk, jnp.maximum(m, msh), m
          )
      src = dst_b
    for g in range(NGRP):
      e0 = g * L
      _, bmask, valid, ndc = _group_masks(dstv, e0, base_node)
      wmask = bmask & valid
      for h in range(N_HEADS):
        m = msc[src, h, pl.ds(8 + e0, L)]
        midx = ndc * N_HEADS + h
        old = plsc.load_gather(lm, [midx])
        plsc.store_scatter(lm, [midx], jnp.maximum(old, m), mask=wmask)
    return 0

  lax.fori_loop(0, nb, block_body, 0)

  def fix_body(i, _):
    o = pl.multiple_of(i * L, L)
    v = lm[pl.ds(o, L)]
    lm[pl.ds(o, L)] = jnp.where(v < -1.0e37, 0.0, v)
    return 0

  lax.fori_loop(0, RN4 // L, fix_body, 0)
  woff = pl.multiple_of(wid * RN4, 8)
  pltpu.sync_copy(lm.at[pl.ds(0, RN4)], maxh.at[pl.ds(woff, RN4)])


def _score_kernel(lgf_hbm, dsth_hbm, eb_hbm, maxh_hbm, scoresh, normh,
                  lgb, dstv, msc, lmv, ls, sb, idxsc, ebv, flag, sem):
  wid = lax.axis_index("c") * NS + lax.axis_index("s")
  pltpu.sync_copy(eb_hbm, ebv)
  aw, ew = _worker_bounds(ebv, wid)
  nb = (ew - aw + B - 1) // B
  base_node = wid * RN
  lanes = _lanes()
  lanes4 = lanes * N_HEADS
  econst = lanes // N_HEADS
  hconst = lanes % N_HEADS

  def init_body(i, _):
    ls[pl.ds(pl.multiple_of(i * L, L), L)] = jnp.zeros((L,), jnp.float32)
    return 0

  lax.fori_loop(0, (RN4 + L) // L, init_body, 0)
  woff = pl.multiple_of(wid * RN4, 8)
  pltpu.sync_copy(maxh_hbm.at[pl.ds(woff, RN4)], lmv.at[pl.ds(0, RN4)])

  def block_body(j, _):
    s = pl.multiple_of(aw + j * B, 8)
    cp1 = pltpu.async_copy(
        lgf_hbm.at[pl.ds(pl.multiple_of(s * N_HEADS, 8), B * N_HEADS)], lgb, sem
    )
    cp2 = pltpu.async_copy(dsth_hbm.at[pl.ds(s, B + L)], dstv, sem)
    cp1.wait()
    cp2.wait()

    zl = jnp.zeros((L,), jnp.int32)
    flag[pl.ds(0, L)] = zl
    for g in range(NGRP):
      e0 = g * L
      _, _, valid, ndc = _group_masks(dstv, e0, base_node)
      plsc.store_scatter(
          flag, [zl], jnp.ones((L,), jnp.int32), mask=jnp.logical_not(valid)
      )
      for h in range(N_HEADS):
        eltidx = lanes4 + (e0 * N_HEADS + h)
        x = plsc.load_gather(lgb, [eltidx])
        midx = ndc * N_HEADS + h
        mv = plsc.load_gather(lmv, [midx])
        sc = jnp.exp(x - mv)
        plsc.store_scatter(sb, [eltidx], sc)
        plsc.addupdate_scatter(ls, [midx], sc, mask=valid)

    clean = flag[pl.ds(0, L)][0] == 0

    @pl.when(clean)
    def _clean():
      pltpu.sync_copy(sb, scoresh.at[pl.ds(pl.multiple_of(s * N_HEADS, 8), B * N_HEADS)])

    @pl.when(jnp.logical_not(clean))
    def _boundary():
      for c in range(NCHUNK):
        for q in range(128 // L):
          evec = econst + (c * (128 // N_HEADS) + q * (L // N_HEADS))
          d4 = plsc.load_gather(dstv, [evec + 8])
          nd4 = d4 - base_node
          v4 = (nd4 >= 0) & (nd4 < RN)
          gidx = (s + evec) * N_HEADS + hconst
          rowv = jnp.where(v4, gidx, DUMMY_ELT + wid * L + lanes)
          idxsc[c, pl.ds(q * L, L)] = rowv
      for c in range(NCHUNK):
        pltpu.async_copy(
            sb.at[pl.ds(c * 128, 128)], scoresh.at[idxsc.at[c]], sem
        ).wait()

    return 0

  lax.fori_loop(0, nb, block_body, 0)
  woff2 = pl.multiple_of(wid * RN4, 8)
  pltpu.sync_copy(ls.at[pl.ds(0, RN4)], normh.at[pl.ds(woff2, RN4)])


def _mesh():
  return plsc.VectorSubcoreMesh(
      core_axis_name="c", subcore_axis_name="s", num_cores=NC, num_subcores=NS
  )


_params = pltpu.CompilerParams(needs_layout_passes=False)

_max_call = pl.kernel(
    _max_kernel,
    out_type=jax.ShapeDtypeStruct((NNP * N_HEADS,), jnp.float32),
    mesh=_mesh(),
    compiler_params=_params,
    scratch_types=[
        pltpu.VMEM((B * N_HEADS,), jnp.float32),
        pltpu.VMEM((B + L,), jnp.int32),
        pltpu.VMEM((2, N_HEADS, B + L), jnp.float32),
        pltpu.VMEM((RN4 + L,), jnp.float32),
        pltpu.VMEM((NW * 2 * L + L,), jnp.int32),
        pltpu.SemaphoreType.DMA,
    ],
)

_score_call = pl.kernel(
    _score_kernel,
    out_type=(
        jax.ShapeDtypeStruct((EPAD * N_HEADS,), jnp.float32),
        jax.ShapeDtypeStruct((NNP * N_HEADS,), jnp.float32),
    ),
    mesh=_mesh(),
    compiler_params=_params,
    scratch_types=[
        pltpu.VMEM((B * N_HEADS,), jnp.float32),
        pltpu.VMEM((B + L,), jnp.int32),
        pltpu.VMEM((2, N_HEADS, B + L), jnp.float32),
        pltpu.VMEM((RN4 + L,), jnp.float32),
        pltpu.VMEM((RN4 + L,), jnp.float32),
        pltpu.VMEM((B * N_HEADS,), jnp.float32),
        pltpu.VMEM((NCHUNK, 128), jnp.int32),
        pltpu.VMEM((NW * 2 * L + L,), jnp.int32),
        pltpu.VMEM((L,), jnp.int32),
        pltpu.SemaphoreType.DMA,
    ],
)


@jax.jit
def kernel(logits, dst):
  lgf = jnp.concatenate([
      logits.reshape(N_EDGES * N_HEADS),
      jnp.zeros(((EPAD - N_EDGES) * N_HEADS,), jnp.float32),
  ])
  dsth = jnp.concatenate([
      jnp.full((8,), -1, jnp.int32),
      dst,
      jnp.full((DLEN - 8 - N_EDGES,), N_NODES, jnp.int32),
  ])
  bounds = jnp.arange(1, NW + 1, dtype=jnp.int32) * RN
  e_hi = jnp.searchsorted(dst, bounds, side="left").astype(jnp.int32)
  e_lo = jnp.concatenate([jnp.zeros((1,), jnp.int32), e_hi[:-1]])
  aw = (e_lo // 8) * 8
  vals = jnp.concatenate([aw, e_hi])
  eb = jnp.concatenate(
      [
          jnp.concatenate(
              [vals[:, None], jnp.zeros((2 * NW, L - 1), jnp.int32)], axis=1
          ).reshape(2 * NW * L),
          jnp.zeros((L,), jnp.int32),
      ]
  )

  maxh = _max_call(lgf, dsth, eb)
  scoresh, normh = _score_call(lgf, dsth, eb, maxh)
  scores = scoresh[: N_EDGES * N_HEADS].reshape(N_EDGES, N_HEADS, 1)
  normalizer = normh[: N_NODES * N_HEADS].reshape(N_NODES, N_HEADS, 1)
  return scores, normalizer
